# baseline (speedup 1.0000x reference)
import jax
import jax.numpy as jnp
from jax.experimental import pallas as pl
from jax.experimental.pallas import tpu as pltpu

C = 1.0
MIN_NORM = 1e-15
EPS = 4e-3


def _expmap0_proj(u):
    un = jnp.maximum(
        jnp.sqrt(jnp.sum(u * u, axis=-1, keepdims=True)), MIN_NORM
    )
    y = jnp.tanh(un) * u / un
    yn = jnp.maximum(
        jnp.sqrt(jnp.sum(y * y, axis=-1, keepdims=True)), MIN_NORM
    )
    maxnorm = 1.0 - EPS
    return jnp.where(yn > maxnorm, y / yn * maxnorm, y)


def _hypagg_kernel(x_ref, at_ref, ab_ref, o_ref, xt_ref):
    @pl.when(pl.program_id(0) == 0)
    def _tangent():
        x = x_ref[...]
        nrm = jnp.maximum(
            jnp.sqrt(jnp.sum(x * x, axis=-1, keepdims=True)), MIN_NORM
        )
        t = jnp.clip(nrm, -1.0 + 1e-7, 1.0 - 1e-7)
        at = 0.5 * (jnp.log1p(t) - jnp.log1p(-t))
        xt_ref[...] = x / nrm * at

    dn = (((1,), (0,)), ((), ()))
    xt = xt_ref[...]
    ut = jax.lax.dot_general(
        at_ref[0], xt, dn,
        preferred_element_type=jnp.float32,
        precision=jax.lax.Precision.DEFAULT,
    )
    ub = jax.lax.dot_general(
        ab_ref[0], xt, dn,
        preferred_element_type=jnp.float32,
        precision=jax.lax.Precision.DEFAULT,
    )
    o_ref[0] = _expmap0_proj(ut)
    o_ref[1] = _expmap0_proj(ub)


def _pick_block(n, candidates):
    for c in candidates:
        if n % c == 0 and c % 8 == 0:
            return c
    return n


def kernel(x, adj):
    n, d = x.shape
    h = n // 2
    bm = _pick_block(h, (200, 256, 128, 80, 64, 40, 16, 8))
    adj2 = adj.reshape(2, h, n)

    out = pl.pallas_call(
        _hypagg_kernel,
        grid=(h // bm,),
        in_specs=[
            pl.BlockSpec((n, d), lambda i: (0, 0)),
            pl.BlockSpec((1, bm, n), lambda i: (0, i, 0)),
            pl.BlockSpec((1, bm, n), lambda i: (1, i, 0)),
        ],
        out_specs=pl.BlockSpec((2, bm, d), lambda i: (0, i, 0)),
        out_shape=jax.ShapeDtypeStruct((2, h, d), jnp.float32),
        scratch_shapes=[pltpu.VMEM((n, d), jnp.float32)],
        compiler_params=pltpu.CompilerParams(
            dimension_semantics=("arbitrary",),
        ),
    )(x, adj2, adj2)
    return out.reshape(n, d)


# speedup vs baseline: 1.0152x; 1.0152x over previous
import jax
import jax.numpy as jnp
from jax.experimental import pallas as pl
from jax.experimental.pallas import tpu as pltpu

C = 1.0
MIN_NORM = 1e-15
EPS = 4e-3


def _hypagg_kernel(x_ref, adj_ref, o_ref, xt_ref):
    @pl.when(pl.program_id(0) == 0)
    def _tangent():
        x = x_ref[...]
        nrm = jnp.maximum(
            jnp.sqrt(jnp.sum(x * x, axis=-1, keepdims=True)), MIN_NORM
        )
        t = jnp.clip(nrm, -1.0 + 1e-7, 1.0 - 1e-7)
        at = 0.5 * (jnp.log1p(t) - jnp.log1p(-t))
        xt_ref[...] = x / nrm * at

    u = jax.lax.dot_general(
        adj_ref[...], xt_ref[...], (((1,), (0,)), ((), ())),
        preferred_element_type=jnp.float32,
        precision=jax.lax.Precision.DEFAULT,
    )
    un = jnp.maximum(
        jnp.sqrt(jnp.sum(u * u, axis=-1, keepdims=True)), MIN_NORM
    )
    y = jnp.tanh(un) * u / un
    yn = jnp.maximum(
        jnp.sqrt(jnp.sum(y * y, axis=-1, keepdims=True)), MIN_NORM
    )
    maxnorm = 1.0 - EPS
    o_ref[...] = jnp.where(yn > maxnorm, y / yn * maxnorm, y)


def _pick_block(n, candidates):
    for c in candidates:
        if n % c == 0 and c % 8 == 0:
            return c
    return n


def kernel(x, adj):
    n, d = x.shape
    bm = _pick_block(n, (400, 512, 256, 200, 128, 80, 64, 40, 16, 8))

    out = pl.pallas_call(
        _hypagg_kernel,
        grid=(n // bm,),
        in_specs=[
            pl.BlockSpec((n, d), lambda i: (0, 0)),
            pl.BlockSpec((bm, n), lambda i: (i, 0)),
        ],
        out_specs=pl.BlockSpec((bm, d), lambda i: (i, 0)),
        out_shape=jax.ShapeDtypeStruct((n, d), jnp.float32),
        scratch_shapes=[pltpu.VMEM((n, d), jnp.float32)],
        compiler_params=pltpu.CompilerParams(
            dimension_semantics=("arbitrary",),
        ),
    )(x, adj)
    return out
